# baseline (speedup 1.0000x reference)
import functools

import jax
import jax.numpy as jnp
from jax import lax
from jax.experimental import pallas as pl
from jax.experimental.pallas import tpu as pltpu
from jax.experimental.pallas import tpu_sc as plsc

EMBED_DIM = 32
BATCH = 16384
NUM_BGCS = 1000000
NUM_DOMAINS = 100000

_INFO = plsc.get_sparse_core_info()
_NC = _INFO.num_cores
_NS = _INFO.num_subcores
_NW = _NC * _NS
_BPW = BATCH // _NW

_CTX_FULL = NUM_BGCS // 128
_DOM_FULL = NUM_DOMAINS // 128
_CTX_ROWS = _CTX_FULL * 128
_DOM_ROWS = _DOM_FULL * 128
_CTX_CPT = 256
_DOM_CPT = 25
_CAP = 1024
_NSC = _CAP // 128
_LCAP = 1344
_WCAP = 192
_PAD_ROWS = 128
_G_ROWS = BATCH + _PAD_ROWS

_LANES = None


def _iota16():
  return jax.lax.iota(jnp.int32, 16)


def _filter_range(idx_v, lo, hi, lst_rb, lst_b, cnts_v):
  lanes = _iota16()
  ng = BATCH // 16

  def pass1(g, c):
    rbv = idx_v[pl.ds(g * 16, 16)]
    m = (rbv >= lo) & (rbv < hi)
    cnts_v[g] = jnp.sum(m.astype(jnp.int32))
    return c

  lax.fori_loop(0, ng, pass1, 0)

  def pass2(g, tot):
    c = cnts_v[g]
    cnts_v[g] = tot
    return tot + c

  n = lax.fori_loop(0, ng, pass2, jnp.int32(0))

  def pass3(g, c):
    rbv = idx_v[pl.ds(g * 16, 16)]
    bv = g * 16 + lanes
    m = (rbv >= lo) & (rbv < hi)
    cur = jnp.minimum(cnts_v[g], _LCAP - 16)
    plsc.store_compressed(lst_rb.at[pl.ds(cur, 16)], rbv, mask=m)
    plsc.store_compressed(lst_b.at[pl.ds(cur, 16)], bv, mask=m)
    return c

  lax.fori_loop(0, ng, pass3, 0)
  n = jnp.minimum(n, _LCAP - 16)
  lst_rb[pl.ds(n, 16)] = jnp.full((16,), jnp.int32(1 << 29), jnp.int32)
  lst_b[pl.ds(n, 16)] = jnp.full((16,), jnp.int32(BATCH), jnp.int32)
  return n


def _phase(tbl_hbm, rows_hbm, maps_hbm, wid, lo_col, hi_col, nwin, wcols,
           last_col, n_lst, lst_rb, lst_b, buf, rowbuf, rb_b, wrb, wb,
           semA, semB, phase_id, cnt_v):
  lanes = _iota16()
  padv = jnp.int32(BATCH) + ((wid * 37 + phase_id * 53 + lanes * 11) & 127)

  def prefill(i, c):
    for h in range(8):
      rb_b[i, pl.ds(h * 16, 16)] = padv
    return c
  lax.fori_loop(0, _NSC, prefill, 0)

  w128 = wcols * 128

  def fire(win, p):
    c0 = jnp.minimum(lo_col + win * wcols, last_col - wcols + 1)
    pltpu.async_copy(tbl_hbm.at[:, :, pl.ds(c0 * 128, w128)],
                     buf.at[p, :, :, pl.ds(0, w128)],
                     semA if p == 0 else semB)

  def drain(p):
    pltpu.make_async_copy(tbl_hbm.at[:, :, pl.ds(0, w128)],
                          buf.at[p, :, :, pl.ds(0, w128)],
                          semA if p == 0 else semB).wait()

  lgroups = (n_lst + 15) // 16

  def wfilter(win):
    c0 = jnp.minimum(lo_col + win * wcols, last_col - wcols + 1)

    def fbody(g, wcur):
      rv = lst_rb[pl.ds(g * 16, 16)]
      bv = lst_b[pl.ds(g * 16, 16)]
      u = (rv >> 7) - c0
      m = (u >= 0) & (u < wcols)
      plsc.store_compressed(wrb.at[pl.ds(wcur, 16)], rv, mask=m)
      plsc.store_compressed(wb.at[pl.ds(wcur, 16)], bv, mask=m)
      return wcur + jnp.sum(m.astype(jnp.int32))

    nw = lax.fori_loop(0, lgroups, fbody, jnp.int32(0))
    nw = jnp.minimum(nw, _WCAP - 16)
    wrb[pl.ds(nw, 16)] = jnp.full((16,), jnp.int32(c0 * 128), jnp.int32)
    wb[pl.ds(nw, 16)] = padv
    return nw

  def extract(win, p, nw, cursor):
    c0 = jnp.minimum(lo_col + win * wcols, last_col - wcols + 1)

    def ebody(j, c):
      rv = wrb[pl.ds(j * 16, 16)]
      bv = wb[pl.ds(j * 16, 16)]
      colv = rv - c0 * 128
      jv = jnp.minimum(c + j * 16 + lanes, _CAP - 1)
      jhi = jv >> 2
      jlo = (jv & 3) * EMBED_DIM
      p16 = jnp.full((16,), jnp.int32(p), jnp.int32)
      for d in range(EMBED_DIM):
        d16 = jnp.full((16,), jnp.int32(d), jnp.int32)
        tr16 = jnp.full((16,), jnp.int32(d // 8), jnp.int32)
        s16 = jnp.full((16,), jnp.int32(d % 8), jnp.int32)
        val = plsc.load_gather(buf, [p16, tr16, s16, colv])
        plsc.store_scatter(rowbuf, [jhi, jlo + d16], val)
      plsc.store_scatter(rb_b, [jv >> 7, jv & 127], bv)
      return c

    lax.fori_loop(0, (nw + 15) // 16, ebody, cursor)
    return jnp.minimum(cursor + nw, _CAP - 16)

  fire(jnp.int32(0), 0)

  def pair(i, cursor):
    fire(2 * i + 1, 1)
    nw = wfilter(2 * i)
    drain(0)
    cursor = extract(2 * i, 0, nw, cursor)
    fire(2 * i + 2, 0)
    nw = wfilter(2 * i + 1)
    drain(1)
    cursor = extract(2 * i + 1, 1, nw, cursor)
    return cursor

  total = lax.fori_loop(0, nwin // 2, pair, jnp.int32(0))
  drain(0)

  plsc.store_scatter(cnt_v, [jnp.full((16,), jnp.int32(phase_id), jnp.int32)],
                     jnp.full((16,), 0, jnp.int32) + total)
  pltpu.sync_copy(rowbuf, rows_hbm.at[pl.ds(wid * (_CAP // 4), _CAP // 4)])
  pltpu.sync_copy(rb_b, maps_hbm.at[pl.ds(wid * _NSC, _NSC)])


def _k1_body(bgc_hbm, dom_hbm, ctx_hbm, t0_hbm, t1_hbm, t2_hbm, t3_hbm,
             cr_hbm, e0r_hbm, e1r_hbm, e2r_hbm, e3r_hbm,
             cm_hbm, e0m_hbm, e1m_hbm, e2m_hbm, e3m_hbm, cnt_hbm,
             idx_v, lst_rb, lst_b, buf, rowbuf, rb_b, wrb, wb, cnt_v,
             cnts_v, semA, semB):
  wid = lax.axis_index("s") * _NC + lax.axis_index("c")

  pltpu.sync_copy(bgc_hbm, idx_v)
  lo_col = wid * _CTX_CPT
  hi_col = jnp.minimum(lo_col + _CTX_CPT, _CTX_FULL)
  n = _filter_range(idx_v, lo_col * 128, hi_col * 128, lst_rb, lst_b, cnts_v)
  _phase(ctx_hbm, cr_hbm, cm_hbm, wid, lo_col, hi_col, _CTX_CPT // 8, 8,
         jnp.int32(_CTX_FULL - 1), n, lst_rb, lst_b, buf, rowbuf, rb_b,
         wrb, wb, semA, semB, 0, cnt_v)

  pltpu.sync_copy(dom_hbm, idx_v)
  lo_col = wid * _DOM_CPT
  hi_col = jnp.minimum(lo_col + _DOM_CPT, _DOM_FULL)
  n = _filter_range(idx_v, lo_col * 128, hi_col * 128, lst_rb, lst_b, cnts_v)
  for t, (tbl, rows, maps) in enumerate((
      (t0_hbm, e0r_hbm, e0m_hbm), (t1_hbm, e1r_hbm, e1m_hbm),
      (t2_hbm, e2r_hbm, e2m_hbm), (t3_hbm, e3r_hbm, e3m_hbm))):
    _phase(tbl, rows, maps, wid, lo_col, hi_col, 8, 4,
           jnp.int32(_DOM_FULL - 1), n, lst_rb, lst_b, buf, rowbuf, rb_b,
           wrb, wb, semA, semB, 1 + t, cnt_v)
  pltpu.sync_copy(cnt_v, cnt_hbm.at[wid])


def _k1b_body(cr_hbm, e0r_hbm, e1r_hbm, e2r_hbm, e3r_hbm,
              cm_hbm, e0m_hbm, e1m_hbm, e2m_hbm, e3m_hbm, cnt_hbm,
              cg_hbm, eg0_hbm, eg1_hbm, eg2_hbm, eg3_hbm,
              rows_v, map_v, cnt_v, sem):
  wid = lax.axis_index("s") * _NC + lax.axis_index("c")
  lanes = _iota16()
  pltpu.sync_copy(cnt_hbm.at[wid], cnt_v)
  cv = cnt_v[pl.ds(0, 16)]
  for t, (rows, maps, out) in enumerate((
      (cr_hbm, cm_hbm, cg_hbm),
      (e0r_hbm, e0m_hbm, eg0_hbm),
      (e1r_hbm, e1m_hbm, eg1_hbm),
      (e2r_hbm, e2m_hbm, eg2_hbm),
      (e3r_hbm, e3m_hbm, eg3_hbm))):
    cnt = jnp.sum(jnp.where(lanes == t, cv, 0))
    pltpu.sync_copy(rows.at[pl.ds(wid * _CAP, _CAP)], rows_v)
    pltpu.sync_copy(maps.at[pl.ds(wid * _NSC, _NSC)], map_v)
    for i in range(_NSC):
      @pl.when(cnt > i * 128)
      def _():
        pltpu.async_copy(
            rows_v.at[pl.ds(i * 128, 128)], out.at[map_v.at[i]], sem)
    for i in range(_NSC):
      @pl.when(cnt > i * 128)
      def _():
        pltpu.make_async_copy(
            rows_v.at[pl.ds(i * 128, 128)], out.at[map_v.at[i]], sem).wait()


def _k2_body(bgc_hbm, dom_hbm, cg_hbm, eg0_hbm, eg1_hbm, eg2_hbm, eg3_hbm,
             ctail_hbm, dt0_hbm, dt1_hbm, dt2_hbm, dt3_hbm, out_hbm,
             bgc_v, dom_v, c_v, e0_v, e1_v, e2_v, e3_v,
             ctail_v, dt0_v, dt1_v, dt2_v, dt3_v, out_v, sem):
  wid = lax.axis_index("s") * _NC + lax.axis_index("c")
  base = wid * _BPW
  lanes = _iota16()

  pltpu.sync_copy(bgc_hbm.at[pl.ds(base, _BPW)], bgc_v)
  pltpu.sync_copy(dom_hbm.at[pl.ds(base, _BPW)], dom_v)
  pltpu.sync_copy(ctail_hbm, ctail_v)
  for src, dst in ((dt0_hbm, dt0_v), (dt1_hbm, dt1_v),
                   (dt2_hbm, dt2_v), (dt3_hbm, dt3_v)):
    pltpu.sync_copy(src, dst)
  copies = [
      pltpu.async_copy(cg_hbm.at[pl.ds(base, _BPW)], c_v, sem),
      pltpu.async_copy(eg0_hbm.at[pl.ds(base, _BPW)], e0_v, sem),
      pltpu.async_copy(eg1_hbm.at[pl.ds(base, _BPW)], e1_v, sem),
      pltpu.async_copy(eg2_hbm.at[pl.ds(base, _BPW)], e2_v, sem),
      pltpu.async_copy(eg3_hbm.at[pl.ds(base, _BPW)], e3_v, sem),
  ]
  for cp in copies:
    cp.wait()

  def patch(g, carry):
    e = g * 16 + lanes
    rbv = bgc_v[pl.ds(g * 16, 16)]
    sbv = dom_v[pl.ds(g * 16, 16)]
    cmask = rbv >= _CTX_ROWS
    smask = sbv >= _DOM_ROWS
    cti = jnp.maximum(rbv - _CTX_ROWS, 0)
    sti = jnp.maximum(sbv - _DOM_ROWS, 0)

    @pl.when(jnp.sum(cmask.astype(jnp.int32)) > 0)
    def _():
      for d in range(EMBED_DIM):
        d16 = jnp.full((16,), jnp.int32(d), jnp.int32)
        val = plsc.load_gather(ctail_v, [cti, d16])
        plsc.store_scatter(c_v, [e, d16], val, mask=cmask)

    @pl.when(jnp.sum(smask.astype(jnp.int32)) > 0)
    def _():
      for tail, ev in ((dt0_v, e0_v), (dt1_v, e1_v),
                       (dt2_v, e2_v), (dt3_v, e3_v)):
        for d in range(EMBED_DIM):
          d16 = jnp.full((16,), jnp.int32(d), jnp.int32)
          val = plsc.load_gather(tail, [sti, d16])
          plsc.store_scatter(ev, [e, d16], val, mask=smask)

    return carry

  lax.fori_loop(0, _BPW // 16, patch, 0)

  def group(g, carry):
    e = g * 16 + lanes
    acc0 = [jnp.zeros((16,), jnp.float32) for _ in range(4)]
    acc1 = [jnp.zeros((16,), jnp.float32) for _ in range(4)]
    for d in range(0, EMBED_DIM, 2):
      da = jnp.full((16,), jnp.int32(d), jnp.int32)
      db = jnp.full((16,), jnp.int32(d + 1), jnp.int32)
      ca = plsc.load_gather(c_v, [e, da])
      cb = plsc.load_gather(c_v, [e, db])
      for t, ev in enumerate((e0_v, e1_v, e2_v, e3_v)):
        acc0[t] = acc0[t] + ca * plsc.load_gather(ev, [e, da])
        acc1[t] = acc1[t] + cb * plsc.load_gather(ev, [e, db])
    for t in range(4):
      plsc.store_scatter(out_v, [e, jnp.full((16,), jnp.int32(t), jnp.int32)],
                         acc0[t] + acc1[t])
    return carry

  lax.fori_loop(0, _BPW // 16, group, 0)
  pltpu.sync_copy(out_v, out_hbm.at[pl.ds(base, _BPW)])


@jax.jit
def kernel(bgc_idx, domain_idx, context_table, domain_table_0,
           domain_table_85, domain_table_170, domain_table_255):
  bgc = bgc_idx.astype(jnp.int32)
  dom = domain_idx.astype(jnp.int32)
  tables = (domain_table_0, domain_table_85, domain_table_170,
            domain_table_255)
  ctx_t = context_table.T.reshape(4, 8, NUM_BGCS)
  tbl_t = tuple(t.T.reshape(4, 8, NUM_DOMAINS) for t in tables)
  ctx_tail = context_table[_CTX_ROWS:]
  dom_tails = tuple(t[_DOM_ROWS:] for t in tables)

  k1 = pl.kernel(
      _k1_body,
      mesh=plsc.VectorSubcoreMesh(core_axis_name="c", subcore_axis_name="s"),
      out_type=(
          tuple(jax.ShapeDtypeStruct((_NW * _CAP // 4, 128), jnp.float32)
                for _ in range(5))
          + tuple(jax.ShapeDtypeStruct((_NW * _NSC, 128), jnp.int32)
                  for _ in range(5))
          + (jax.ShapeDtypeStruct((_NW, 16), jnp.int32),)),
      compiler_params=pltpu.CompilerParams(
          needs_layout_passes=False, use_tc_tiling_on_sc=True),
      scratch_types=[
          pltpu.VMEM((BATCH,), jnp.int32),
          pltpu.VMEM((_LCAP,), jnp.int32),
          pltpu.VMEM((_LCAP,), jnp.int32),
          pltpu.VMEM((2, 4, 8, 1024), jnp.float32),
          pltpu.VMEM((_CAP // 4, 128), jnp.float32),
          pltpu.VMEM((_NSC, 128), jnp.int32),
          pltpu.VMEM((_WCAP,), jnp.int32),
          pltpu.VMEM((_WCAP,), jnp.int32),
          pltpu.VMEM((16,), jnp.int32),
          pltpu.SMEM((BATCH // 16,), jnp.int32),
          pltpu.SemaphoreType.DMA,
          pltpu.SemaphoreType.DMA,
      ],
  )
  rows_maps = k1(bgc, dom, ctx_t, *tbl_t)
  rows_maps = tuple(
      r.reshape(_NW * _CAP, EMBED_DIM) if i < 5 else r
      for i, r in enumerate(rows_maps))

  k1b = pl.kernel(
      _k1b_body,
      mesh=plsc.VectorSubcoreMesh(core_axis_name="c", subcore_axis_name="s"),
      out_type=tuple(jax.ShapeDtypeStruct((_G_ROWS, EMBED_DIM), jnp.float32)
                     for _ in range(5)),
      compiler_params=pltpu.CompilerParams(
          needs_layout_passes=False, use_tc_tiling_on_sc=False),
      scratch_types=[
          pltpu.VMEM((_CAP, EMBED_DIM), jnp.float32),
          pltpu.VMEM((_NSC, 128), jnp.int32),
          pltpu.VMEM((16,), jnp.int32),
          pltpu.SemaphoreType.DMA,
      ],
  )
  cg, eg0, eg1, eg2, eg3 = k1b(*rows_maps)

  k2 = pl.kernel(
      _k2_body,
      mesh=plsc.VectorSubcoreMesh(core_axis_name="c", subcore_axis_name="s"),
      out_type=jax.ShapeDtypeStruct((BATCH, 4), jnp.float32),
      compiler_params=pltpu.CompilerParams(
          needs_layout_passes=False, use_tc_tiling_on_sc=False),
      scratch_types=[
          pltpu.VMEM((_BPW,), jnp.int32),
          pltpu.VMEM((_BPW,), jnp.int32),
          pltpu.VMEM((_BPW, EMBED_DIM), jnp.float32),
          pltpu.VMEM((_BPW, EMBED_DIM), jnp.float32),
          pltpu.VMEM((_BPW, EMBED_DIM), jnp.float32),
          pltpu.VMEM((_BPW, EMBED_DIM), jnp.float32),
          pltpu.VMEM((_BPW, EMBED_DIM), jnp.float32),
          pltpu.VMEM((64, EMBED_DIM), jnp.float32),
          pltpu.VMEM((32, EMBED_DIM), jnp.float32),
          pltpu.VMEM((32, EMBED_DIM), jnp.float32),
          pltpu.VMEM((32, EMBED_DIM), jnp.float32),
          pltpu.VMEM((32, EMBED_DIM), jnp.float32),
          pltpu.VMEM((_BPW, 4), jnp.float32),
          pltpu.SemaphoreType.DMA,
      ],
  )
  return k2(bgc, dom, cg, eg0, eg1, eg2, eg3, ctx_tail, *dom_tails)


# speedup vs baseline: 1.0222x; 1.0222x over previous
import functools

import jax
import jax.numpy as jnp
from jax import lax
from jax.experimental import pallas as pl
from jax.experimental.pallas import tpu as pltpu
from jax.experimental.pallas import tpu_sc as plsc

EMBED_DIM = 32
BATCH = 16384
NUM_BGCS = 1000000
NUM_DOMAINS = 100000

_INFO = plsc.get_sparse_core_info()
_NC = _INFO.num_cores
_NS = _INFO.num_subcores
_NW = _NC * _NS
_BPW = BATCH // _NW

_CTX_FULL = NUM_BGCS // 128
_DOM_FULL = NUM_DOMAINS // 128
_CTX_ROWS = _CTX_FULL * 128
_DOM_ROWS = _DOM_FULL * 128
_CTX_CPT = 256
_DOM_CPT = 25
_CAP = 1024
_NSC = _CAP // 128
_LCAP = 1344
_WCAP = 192
_PAD_ROWS = 128
_G_ROWS = BATCH + _PAD_ROWS

_LANES = None


def _iota16():
  return jax.lax.iota(jnp.int32, 16)


def _filter_range(idx_v, lo, hi, lst_rb, lst_b):
  lanes = _iota16()

  def body(g, cur):
    for h in range(2):
      rbv = idx_v[pl.ds(g * 32 + h * 16, 16)]
      bv = g * 32 + h * 16 + lanes
      m = (rbv >= lo) & (rbv < hi)
      plsc.store_compressed(lst_rb.at[pl.ds(cur, 16)], rbv, mask=m)
      plsc.store_compressed(lst_b.at[pl.ds(cur, 16)], bv, mask=m)
      cur = cur + jnp.sum(m.astype(jnp.int32))
    return cur

  n = lax.fori_loop(0, BATCH // 32, body, jnp.int32(0))
  n = jnp.minimum(n, _LCAP - 16)
  lst_rb[pl.ds(n, 16)] = jnp.full((16,), jnp.int32(1 << 29), jnp.int32)
  lst_b[pl.ds(n, 16)] = jnp.full((16,), jnp.int32(BATCH), jnp.int32)
  return n


def _phase(tbl_hbm, rows_hbm, maps_hbm, wid, lo_col, hi_col, nwin, wcols,
           last_col, n_lst, lst_rb, lst_b, buf, rowbuf, rb_b, wrb, wb,
           semA, semB, phase_id, cnt_v):
  lanes = _iota16()
  padv = jnp.int32(BATCH) + ((wid * 37 + phase_id * 53 + lanes * 11) & 127)

  def prefill(i, c):
    for h in range(8):
      rb_b[i, pl.ds(h * 16, 16)] = padv
    return c
  lax.fori_loop(0, _NSC, prefill, 0)

  w128 = wcols * 128

  def fire(win, p):
    c0 = jnp.minimum(lo_col + win * wcols, last_col - wcols + 1)
    pltpu.async_copy(tbl_hbm.at[:, :, pl.ds(c0 * 128, w128)],
                     buf.at[p, :, :, pl.ds(0, w128)],
                     semA if p == 0 else semB)

  def drain(p):
    pltpu.make_async_copy(tbl_hbm.at[:, :, pl.ds(0, w128)],
                          buf.at[p, :, :, pl.ds(0, w128)],
                          semA if p == 0 else semB).wait()

  lgroups = (n_lst + 15) // 16

  def wfilter(win):
    c0 = jnp.minimum(lo_col + win * wcols, last_col - wcols + 1)

    def fbody(g, wcur):
      rv = lst_rb[pl.ds(g * 16, 16)]
      bv = lst_b[pl.ds(g * 16, 16)]
      u = (rv >> 7) - c0
      m = (u >= 0) & (u < wcols)
      plsc.store_compressed(wrb.at[pl.ds(wcur, 16)], rv, mask=m)
      plsc.store_compressed(wb.at[pl.ds(wcur, 16)], bv, mask=m)
      return wcur + jnp.sum(m.astype(jnp.int32))

    nw = lax.fori_loop(0, lgroups, fbody, jnp.int32(0))
    nw = jnp.minimum(nw, _WCAP - 16)
    wrb[pl.ds(nw, 16)] = jnp.full((16,), jnp.int32(c0 * 128), jnp.int32)
    wb[pl.ds(nw, 16)] = padv
    return nw

  def extract(win, p, nw, cursor):
    c0 = jnp.minimum(lo_col + win * wcols, last_col - wcols + 1)

    def ebody(j, c):
      rv = wrb[pl.ds(j * 16, 16)]
      bv = wb[pl.ds(j * 16, 16)]
      colv = rv - c0 * 128
      jv = jnp.minimum(c + j * 16 + lanes, _CAP - 1)
      jhi = jv >> 2
      jlo = (jv & 3) * EMBED_DIM
      p16 = jnp.full((16,), jnp.int32(p), jnp.int32)
      for d in range(EMBED_DIM):
        d16 = jnp.full((16,), jnp.int32(d), jnp.int32)
        tr16 = jnp.full((16,), jnp.int32(d // 8), jnp.int32)
        s16 = jnp.full((16,), jnp.int32(d % 8), jnp.int32)
        val = plsc.load_gather(buf, [p16, tr16, s16, colv])
        plsc.store_scatter(rowbuf, [jhi, jlo + d16], val)
      plsc.store_scatter(rb_b, [jv >> 7, jv & 127], bv)
      return c

    lax.fori_loop(0, (nw + 15) // 16, ebody, cursor)
    return jnp.minimum(cursor + nw, _CAP - 16)

  fire(jnp.int32(0), 0)

  def pair(i, cursor):
    fire(2 * i + 1, 1)
    nw = wfilter(2 * i)
    drain(0)
    cursor = extract(2 * i, 0, nw, cursor)
    fire(2 * i + 2, 0)
    nw = wfilter(2 * i + 1)
    drain(1)
    cursor = extract(2 * i + 1, 1, nw, cursor)
    return cursor

  total = lax.fori_loop(0, nwin // 2, pair, jnp.int32(0))
  drain(0)

  plsc.store_scatter(cnt_v, [jnp.full((16,), jnp.int32(phase_id), jnp.int32)],
                     jnp.full((16,), 0, jnp.int32) + total)
  pltpu.sync_copy(rowbuf, rows_hbm.at[pl.ds(wid * (_CAP // 4), _CAP // 4)])
  pltpu.sync_copy(rb_b, maps_hbm.at[pl.ds(wid * _NSC, _NSC)])


def _k1_body(bgc_hbm, dom_hbm, ctx_hbm, t0_hbm, t1_hbm, t2_hbm, t3_hbm,
             cr_hbm, e0r_hbm, e1r_hbm, e2r_hbm, e3r_hbm,
             cm_hbm, e0m_hbm, e1m_hbm, e2m_hbm, e3m_hbm, cnt_hbm,
             idx_v, lst_rb, lst_b, buf, rowbuf, rb_b, wrb, wb, cnt_v,
             semA, semB):
  wid = lax.axis_index("s") * _NC + lax.axis_index("c")

  pltpu.sync_copy(bgc_hbm, idx_v)
  lo_col = wid * _CTX_CPT
  hi_col = jnp.minimum(lo_col + _CTX_CPT, _CTX_FULL)
  n = _filter_range(idx_v, lo_col * 128, hi_col * 128, lst_rb, lst_b)
  _phase(ctx_hbm, cr_hbm, cm_hbm, wid, lo_col, hi_col, _CTX_CPT // 8, 8,
         jnp.int32(_CTX_FULL - 1), n, lst_rb, lst_b, buf, rowbuf, rb_b,
         wrb, wb, semA, semB, 0, cnt_v)

  pltpu.sync_copy(dom_hbm, idx_v)
  lo_col = wid * _DOM_CPT
  hi_col = jnp.minimum(lo_col + _DOM_CPT, _DOM_FULL)
  n = _filter_range(idx_v, lo_col * 128, hi_col * 128, lst_rb, lst_b)
  for t, (tbl, rows, maps) in enumerate((
      (t0_hbm, e0r_hbm, e0m_hbm), (t1_hbm, e1r_hbm, e1m_hbm),
      (t2_hbm, e2r_hbm, e2m_hbm), (t3_hbm, e3r_hbm, e3m_hbm))):
    _phase(tbl, rows, maps, wid, lo_col, hi_col, 8, 4,
           jnp.int32(_DOM_FULL - 1), n, lst_rb, lst_b, buf, rowbuf, rb_b,
           wrb, wb, semA, semB, 1 + t, cnt_v)
  pltpu.sync_copy(cnt_v, cnt_hbm.at[wid])


def _k1b_body(cr_hbm, e0r_hbm, e1r_hbm, e2r_hbm, e3r_hbm,
              cm_hbm, e0m_hbm, e1m_hbm, e2m_hbm, e3m_hbm, cnt_hbm,
              cg_hbm, eg0_hbm, eg1_hbm, eg2_hbm, eg3_hbm,
              rows_v, map_v, cnt_v, sem):
  wid = lax.axis_index("s") * _NC + lax.axis_index("c")
  lanes = _iota16()
  pltpu.sync_copy(cnt_hbm.at[wid], cnt_v)
  cv = cnt_v[pl.ds(0, 16)]
  for t, (rows, maps, out) in enumerate((
      (cr_hbm, cm_hbm, cg_hbm),
      (e0r_hbm, e0m_hbm, eg0_hbm),
      (e1r_hbm, e1m_hbm, eg1_hbm),
      (e2r_hbm, e2m_hbm, eg2_hbm),
      (e3r_hbm, e3m_hbm, eg3_hbm))):
    cnt = jnp.sum(jnp.where(lanes == t, cv, 0))
    pltpu.sync_copy(rows.at[pl.ds(wid * _CAP, _CAP)], rows_v)
    pltpu.sync_copy(maps.at[pl.ds(wid * _NSC, _NSC)], map_v)
    for i in range(_NSC):
      @pl.when(cnt > i * 128)
      def _():
        pltpu.async_copy(
            rows_v.at[pl.ds(i * 128, 128)], out.at[map_v.at[i]], sem)
    for i in range(_NSC):
      @pl.when(cnt > i * 128)
      def _():
        pltpu.make_async_copy(
            rows_v.at[pl.ds(i * 128, 128)], out.at[map_v.at[i]], sem).wait()


def _k2_body(bgc_hbm, dom_hbm, cg_hbm, eg0_hbm, eg1_hbm, eg2_hbm, eg3_hbm,
             ctail_hbm, dt0_hbm, dt1_hbm, dt2_hbm, dt3_hbm, out_hbm,
             bgc_v, dom_v, c_v, e0_v, e1_v, e2_v, e3_v,
             ctail_v, dt0_v, dt1_v, dt2_v, dt3_v, out_v, sem):
  wid = lax.axis_index("s") * _NC + lax.axis_index("c")
  base = wid * _BPW
  lanes = _iota16()

  pltpu.sync_copy(bgc_hbm.at[pl.ds(base, _BPW)], bgc_v)
  pltpu.sync_copy(dom_hbm.at[pl.ds(base, _BPW)], dom_v)
  pltpu.sync_copy(ctail_hbm, ctail_v)
  for src, dst in ((dt0_hbm, dt0_v), (dt1_hbm, dt1_v),
                   (dt2_hbm, dt2_v), (dt3_hbm, dt3_v)):
    pltpu.sync_copy(src, dst)
  copies = [
      pltpu.async_copy(cg_hbm.at[pl.ds(base, _BPW)], c_v, sem),
      pltpu.async_copy(eg0_hbm.at[pl.ds(base, _BPW)], e0_v, sem),
      pltpu.async_copy(eg1_hbm.at[pl.ds(base, _BPW)], e1_v, sem),
      pltpu.async_copy(eg2_hbm.at[pl.ds(base, _BPW)], e2_v, sem),
      pltpu.async_copy(eg3_hbm.at[pl.ds(base, _BPW)], e3_v, sem),
  ]
  for cp in copies:
    cp.wait()

  def patch(g, carry):
    e = g * 16 + lanes
    rbv = bgc_v[pl.ds(g * 16, 16)]
    sbv = dom_v[pl.ds(g * 16, 16)]
    cmask = rbv >= _CTX_ROWS
    smask = sbv >= _DOM_ROWS
    cti = jnp.maximum(rbv - _CTX_ROWS, 0)
    sti = jnp.maximum(sbv - _DOM_ROWS, 0)

    @pl.when(jnp.sum(cmask.astype(jnp.int32)) > 0)
    def _():
      for d in range(EMBED_DIM):
        d16 = jnp.full((16,), jnp.int32(d), jnp.int32)
        val = plsc.load_gather(ctail_v, [cti, d16])
        plsc.store_scatter(c_v, [e, d16], val, mask=cmask)

    @pl.when(jnp.sum(smask.astype(jnp.int32)) > 0)
    def _():
      for tail, ev in ((dt0_v, e0_v), (dt1_v, e1_v),
                       (dt2_v, e2_v), (dt3_v, e3_v)):
        for d in range(EMBED_DIM):
          d16 = jnp.full((16,), jnp.int32(d), jnp.int32)
          val = plsc.load_gather(tail, [sti, d16])
          plsc.store_scatter(ev, [e, d16], val, mask=smask)

    return carry

  lax.fori_loop(0, _BPW // 16, patch, 0)

  def group(g, carry):
    e = g * 16 + lanes
    accs = [jnp.zeros((16,), jnp.float32) for _ in range(4)]
    for d in range(EMBED_DIM):
      d16 = jnp.full((16,), jnp.int32(d), jnp.int32)
      cd = plsc.load_gather(c_v, [e, d16])
      for t, ev in enumerate((e0_v, e1_v, e2_v, e3_v)):
        accs[t] = accs[t] + cd * plsc.load_gather(ev, [e, d16])
    for t in range(4):
      plsc.store_scatter(out_v, [e, jnp.full((16,), jnp.int32(t), jnp.int32)],
                         accs[t])
    return carry

  lax.fori_loop(0, _BPW // 16, group, 0)
  pltpu.sync_copy(out_v, out_hbm.at[pl.ds(base, _BPW)])


@jax.jit
def kernel(bgc_idx, domain_idx, context_table, domain_table_0,
           domain_table_85, domain_table_170, domain_table_255):
  bgc = bgc_idx.astype(jnp.int32)
  dom = domain_idx.astype(jnp.int32)
  tables = (domain_table_0, domain_table_85, domain_table_170,
            domain_table_255)
  ctx_t = context_table.T.reshape(4, 8, NUM_BGCS)
  tbl_t = tuple(t.T.reshape(4, 8, NUM_DOMAINS) for t in tables)
  ctx_tail = context_table[_CTX_ROWS:]
  dom_tails = tuple(t[_DOM_ROWS:] for t in tables)

  k1 = pl.kernel(
      _k1_body,
      mesh=plsc.VectorSubcoreMesh(core_axis_name="c", subcore_axis_name="s"),
      out_type=(
          tuple(jax.ShapeDtypeStruct((_NW * _CAP // 4, 128), jnp.float32)
                for _ in range(5))
          + tuple(jax.ShapeDtypeStruct((_NW * _NSC, 128), jnp.int32)
                  for _ in range(5))
          + (jax.ShapeDtypeStruct((_NW, 16), jnp.int32),)),
      compiler_params=pltpu.CompilerParams(
          needs_layout_passes=False, use_tc_tiling_on_sc=True),
      scratch_types=[
          pltpu.VMEM((BATCH,), jnp.int32),
          pltpu.VMEM((_LCAP,), jnp.int32),
          pltpu.VMEM((_LCAP,), jnp.int32),
          pltpu.VMEM((2, 4, 8, 1024), jnp.float32),
          pltpu.VMEM((_CAP // 4, 128), jnp.float32),
          pltpu.VMEM((_NSC, 128), jnp.int32),
          pltpu.VMEM((_WCAP,), jnp.int32),
          pltpu.VMEM((_WCAP,), jnp.int32),
          pltpu.VMEM((16,), jnp.int32),
          pltpu.SemaphoreType.DMA,
          pltpu.SemaphoreType.DMA,
      ],
  )
  rows_maps = k1(bgc, dom, ctx_t, *tbl_t)
  rows_maps = tuple(
      r.reshape(_NW * _CAP, EMBED_DIM) if i < 5 else r
      for i, r in enumerate(rows_maps))

  k1b = pl.kernel(
      _k1b_body,
      mesh=plsc.VectorSubcoreMesh(core_axis_name="c", subcore_axis_name="s"),
      out_type=tuple(jax.ShapeDtypeStruct((_G_ROWS, EMBED_DIM), jnp.float32)
                     for _ in range(5)),
      compiler_params=pltpu.CompilerParams(
          needs_layout_passes=False, use_tc_tiling_on_sc=False),
      scratch_types=[
          pltpu.VMEM((_CAP, EMBED_DIM), jnp.float32),
          pltpu.VMEM((_NSC, 128), jnp.int32),
          pltpu.VMEM((16,), jnp.int32),
          pltpu.SemaphoreType.DMA,
      ],
  )
  cg, eg0, eg1, eg2, eg3 = k1b(*rows_maps)

  k2 = pl.kernel(
      _k2_body,
      mesh=plsc.VectorSubcoreMesh(core_axis_name="c", subcore_axis_name="s"),
      out_type=jax.ShapeDtypeStruct((BATCH, 4), jnp.float32),
      compiler_params=pltpu.CompilerParams(
          needs_layout_passes=False, use_tc_tiling_on_sc=False),
      scratch_types=[
          pltpu.VMEM((_BPW,), jnp.int32),
          pltpu.VMEM((_BPW,), jnp.int32),
          pltpu.VMEM((_BPW, EMBED_DIM), jnp.float32),
          pltpu.VMEM((_BPW, EMBED_DIM), jnp.float32),
          pltpu.VMEM((_BPW, EMBED_DIM), jnp.float32),
          pltpu.VMEM((_BPW, EMBED_DIM), jnp.float32),
          pltpu.VMEM((_BPW, EMBED_DIM), jnp.float32),
          pltpu.VMEM((64, EMBED_DIM), jnp.float32),
          pltpu.VMEM((32, EMBED_DIM), jnp.float32),
          pltpu.VMEM((32, EMBED_DIM), jnp.float32),
          pltpu.VMEM((32, EMBED_DIM), jnp.float32),
          pltpu.VMEM((32, EMBED_DIM), jnp.float32),
          pltpu.VMEM((_BPW, 4), jnp.float32),
          pltpu.SemaphoreType.DMA,
      ],
  )
  return k2(bgc, dom, cg, eg0, eg1, eg2, eg3, ctx_tail, *dom_tails)


# speedup vs baseline: 1.1492x; 1.1243x over previous
import functools

import jax
import jax.numpy as jnp
from jax import lax
from jax.experimental import pallas as pl
from jax.experimental.pallas import tpu as pltpu
from jax.experimental.pallas import tpu_sc as plsc

EMBED_DIM = 32
BATCH = 16384
NUM_BGCS = 1000000
NUM_DOMAINS = 100000

_INFO = plsc.get_sparse_core_info()
_NC = _INFO.num_cores
_NS = _INFO.num_subcores
_NW = _NC * _NS
_BPW = BATCH // _NW

_CTX_FULL = NUM_BGCS // 128
_DOM_FULL = NUM_DOMAINS // 128
_CTX_ROWS = _CTX_FULL * 128
_DOM_ROWS = _DOM_FULL * 128
_CTX_CPT = 256
_DOM_CPT = 25
_CAP = 1024
_NSC = _CAP // 128
_LCAP = 1344
_WCAP = 192
_PAD_ROWS = 128
_G_ROWS = BATCH + _PAD_ROWS

_LANES = None


def _iota16():
  return jax.lax.iota(jnp.int32, 16)


def _filter_range(idx_v, lo, hi, lst_rb, lst_b):
  lanes = _iota16()

  def body(g, cur):
    for h in range(2):
      rbv = idx_v[pl.ds(g * 32 + h * 16, 16)]
      bv = g * 32 + h * 16 + lanes
      m = (rbv >= lo) & (rbv < hi)
      plsc.store_compressed(lst_rb.at[pl.ds(cur, 16)], rbv, mask=m)
      plsc.store_compressed(lst_b.at[pl.ds(cur, 16)], bv, mask=m)
      cur = cur + jnp.sum(m.astype(jnp.int32))
    return cur

  n = lax.fori_loop(0, BATCH // 32, body, jnp.int32(0))
  n = jnp.minimum(n, _LCAP - 16)
  lst_rb[pl.ds(n, 16)] = jnp.full((16,), jnp.int32(1 << 29), jnp.int32)
  lst_b[pl.ds(n, 16)] = jnp.full((16,), jnp.int32(BATCH), jnp.int32)
  return n


def _phase(tbl_hbm, rows_hbm, maps_hbm, wid, lo_col, hi_col, nwin, wcols,
           last_col, n_lst, lst_rb, lst_b, buf, rowbuf, rb_b, wrb, wb,
           semA, semB, phase_id, cnt_v):
  lanes = _iota16()
  padv = jnp.int32(BATCH) + ((wid * 37 + phase_id * 53 + lanes * 11) & 127)

  def prefill(i, c):
    for h in range(8):
      rb_b[i, pl.ds(h * 16, 16)] = padv
    return c
  lax.fori_loop(0, _NSC, prefill, 0)

  w128 = wcols * 128

  def fire(win, p):
    c0 = jnp.minimum(lo_col + win * wcols, last_col - wcols + 1)
    pltpu.async_copy(tbl_hbm.at[:, :, pl.ds(c0 * 128, w128)],
                     buf.at[p, :, :, pl.ds(0, w128)],
                     semA if p == 0 else semB)

  def drain(p):
    pltpu.make_async_copy(tbl_hbm.at[:, :, pl.ds(0, w128)],
                          buf.at[p, :, :, pl.ds(0, w128)],
                          semA if p == 0 else semB).wait()

  lgroups = (n_lst + 15) // 16

  def wfilter(win):
    c0 = jnp.minimum(lo_col + win * wcols, last_col - wcols + 1)

    def fbody(g, wcur):
      rv = lst_rb[pl.ds(g * 16, 16)]
      bv = lst_b[pl.ds(g * 16, 16)]
      u = (rv >> 7) - c0
      m = (u >= 0) & (u < wcols)
      plsc.store_compressed(wrb.at[pl.ds(wcur, 16)], rv, mask=m)
      plsc.store_compressed(wb.at[pl.ds(wcur, 16)], bv, mask=m)
      return wcur + jnp.sum(m.astype(jnp.int32))

    nw = lax.fori_loop(0, lgroups, fbody, jnp.int32(0))
    nw = jnp.minimum(nw, _WCAP - 16)
    wrb[pl.ds(nw, 16)] = jnp.full((16,), jnp.int32(c0 * 128), jnp.int32)
    wb[pl.ds(nw, 16)] = padv
    return nw

  def extract(win, p, nw, cursor):
    c0 = jnp.minimum(lo_col + win * wcols, last_col - wcols + 1)

    def ebody(j, c):
      rv = wrb[pl.ds(j * 16, 16)]
      bv = wb[pl.ds(j * 16, 16)]
      colv = rv - c0 * 128
      jv = jnp.minimum(c + j * 16 + lanes, _CAP - 1)
      jhi = jv >> 2
      jlo = (jv & 3) * EMBED_DIM
      p16 = jnp.full((16,), jnp.int32(p), jnp.int32)
      for d in range(EMBED_DIM):
        d16 = jnp.full((16,), jnp.int32(d), jnp.int32)
        tr16 = jnp.full((16,), jnp.int32(d // 8), jnp.int32)
        s16 = jnp.full((16,), jnp.int32(d % 8), jnp.int32)
        val = plsc.load_gather(buf, [p16, tr16, s16, colv])
        plsc.store_scatter(rowbuf, [jhi, jlo + d16], val)
      plsc.store_scatter(rb_b, [jv >> 7, jv & 127], bv)
      return c

    lax.fori_loop(0, (nw + 15) // 16, ebody, cursor)
    return jnp.minimum(cursor + nw, _CAP - 16)

  fire(jnp.int32(0), 0)

  def pair(i, cursor):
    fire(2 * i + 1, 1)
    nw = wfilter(2 * i)
    drain(0)
    cursor = extract(2 * i, 0, nw, cursor)
    fire(2 * i + 2, 0)
    nw = wfilter(2 * i + 1)
    drain(1)
    cursor = extract(2 * i + 1, 1, nw, cursor)
    return cursor

  total = lax.fori_loop(0, nwin // 2, pair, jnp.int32(0))
  drain(0)

  plsc.store_scatter(cnt_v, [jnp.full((16,), jnp.int32(phase_id), jnp.int32)],
                     jnp.full((16,), 0, jnp.int32) + total)
  pltpu.sync_copy(rowbuf, rows_hbm.at[pl.ds(wid * (_CAP // 4), _CAP // 4)])
  pltpu.sync_copy(rb_b, maps_hbm.at[pl.ds(wid * _NSC, _NSC)])


def _k1_body(bgc_hbm, dom_hbm, ctx_hbm, t0_hbm, t1_hbm, t2_hbm, t3_hbm,
             cr_hbm, e0r_hbm, e1r_hbm, e2r_hbm, e3r_hbm,
             cm_hbm, e0m_hbm, e1m_hbm, e2m_hbm, e3m_hbm, cnt_hbm,
             idx_v, lst_rb, lst_b, buf, rowbuf, rb_b, wrb, wb, cnt_v,
             semA, semB):
  wid = lax.axis_index("s") * _NC + lax.axis_index("c")

  pltpu.sync_copy(bgc_hbm, idx_v)
  lo_col = wid * _CTX_CPT
  hi_col = jnp.minimum(lo_col + _CTX_CPT, _CTX_FULL)
  n = _filter_range(idx_v, lo_col * 128, hi_col * 128, lst_rb, lst_b)
  _phase(ctx_hbm, cr_hbm, cm_hbm, wid, lo_col, hi_col, _CTX_CPT // 8, 8,
         jnp.int32(_CTX_FULL - 1), n, lst_rb, lst_b, buf, rowbuf, rb_b,
         wrb, wb, semA, semB, 0, cnt_v)

  pltpu.sync_copy(dom_hbm, idx_v)
  lo_col = wid * _DOM_CPT
  hi_col = jnp.minimum(lo_col + _DOM_CPT, _DOM_FULL)
  n = _filter_range(idx_v, lo_col * 128, hi_col * 128, lst_rb, lst_b)
  for t, (tbl, rows, maps) in enumerate((
      (t0_hbm, e0r_hbm, e0m_hbm), (t1_hbm, e1r_hbm, e1m_hbm),
      (t2_hbm, e2r_hbm, e2m_hbm), (t3_hbm, e3r_hbm, e3m_hbm))):
    _phase(tbl, rows, maps, wid, lo_col, hi_col, 8, 4,
           jnp.int32(_DOM_FULL - 1), n, lst_rb, lst_b, buf, rowbuf, rb_b,
           wrb, wb, semA, semB, 1 + t, cnt_v)
  pltpu.sync_copy(cnt_v, cnt_hbm.at[wid])


def _k1b_body(cr_hbm, e0r_hbm, e1r_hbm, e2r_hbm, e3r_hbm,
              cm_hbm, e0m_hbm, e1m_hbm, e2m_hbm, e3m_hbm, cnt_hbm,
              cg_hbm, eg0_hbm, eg1_hbm, eg2_hbm, eg3_hbm,
              rows_v, map_v, cnt_v, sem):
  wid = lax.axis_index("s") * _NC + lax.axis_index("c")
  lanes = _iota16()
  pltpu.sync_copy(cnt_hbm.at[wid], cnt_v)
  cv = cnt_v[pl.ds(0, 16)]
  for t, (rows, maps, out) in enumerate((
      (cr_hbm, cm_hbm, cg_hbm),
      (e0r_hbm, e0m_hbm, eg0_hbm),
      (e1r_hbm, e1m_hbm, eg1_hbm),
      (e2r_hbm, e2m_hbm, eg2_hbm),
      (e3r_hbm, e3m_hbm, eg3_hbm))):
    cnt = jnp.sum(jnp.where(lanes == t, cv, 0))
    pltpu.sync_copy(rows.at[pl.ds(wid * _CAP, _CAP)], rows_v)
    pltpu.sync_copy(maps.at[pl.ds(wid * _NSC, _NSC)], map_v)
    for i in range(_NSC):
      @pl.when(cnt > i * 128)
      def _():
        pltpu.async_copy(
            rows_v.at[pl.ds(i * 128, 128)], out.at[map_v.at[i]], sem)
    for i in range(_NSC):
      @pl.when(cnt > i * 128)
      def _():
        pltpu.make_async_copy(
            rows_v.at[pl.ds(i * 128, 128)], out.at[map_v.at[i]], sem).wait()


def _k2_body(bgc_hbm, dom_hbm, cg_hbm, eg0_hbm, eg1_hbm, eg2_hbm, eg3_hbm,
             ctail_hbm, dt0_hbm, dt1_hbm, dt2_hbm, dt3_hbm, out_hbm,
             bgc_v, dom_v, c_v, e0_v, e1_v, e2_v, e3_v,
             ctail_v, dt0_v, dt1_v, dt2_v, dt3_v, out_flat, sem):
  wid = lax.axis_index("s") * _NC + lax.axis_index("c")
  base = wid * _BPW
  lanes = _iota16()

  pltpu.sync_copy(bgc_hbm.at[pl.ds(base, _BPW)], bgc_v)
  pltpu.sync_copy(dom_hbm.at[pl.ds(base, _BPW)], dom_v)
  pltpu.sync_copy(ctail_hbm, ctail_v)
  for src, dst in ((dt0_hbm, dt0_v), (dt1_hbm, dt1_v),
                   (dt2_hbm, dt2_v), (dt3_hbm, dt3_v)):
    pltpu.sync_copy(src, dst)
  copies = [
      pltpu.async_copy(cg_hbm.at[pl.ds(base, _BPW)], c_v, sem),
      pltpu.async_copy(eg0_hbm.at[pl.ds(base, _BPW)], e0_v, sem),
      pltpu.async_copy(eg1_hbm.at[pl.ds(base, _BPW)], e1_v, sem),
      pltpu.async_copy(eg2_hbm.at[pl.ds(base, _BPW)], e2_v, sem),
      pltpu.async_copy(eg3_hbm.at[pl.ds(base, _BPW)], e3_v, sem),
  ]
  for cp in copies:
    cp.wait()

  def patch(g, carry):
    e = g * 16 + lanes
    rbv = bgc_v[pl.ds(g * 16, 16)]
    sbv = dom_v[pl.ds(g * 16, 16)]
    cmask = rbv >= _CTX_ROWS
    smask = sbv >= _DOM_ROWS
    cti = jnp.maximum(rbv - _CTX_ROWS, 0)
    sti = jnp.maximum(sbv - _DOM_ROWS, 0)

    @pl.when(jnp.sum(cmask.astype(jnp.int32)) > 0)
    def _():
      for d in range(EMBED_DIM):
        d16 = jnp.full((16,), jnp.int32(d), jnp.int32)
        val = plsc.load_gather(ctail_v, [cti, d16])
        plsc.store_scatter(c_v, [e, d16], val, mask=cmask)

    @pl.when(jnp.sum(smask.astype(jnp.int32)) > 0)
    def _():
      for tail, ev in ((dt0_v, e0_v), (dt1_v, e1_v),
                       (dt2_v, e2_v), (dt3_v, e3_v)):
        for d in range(EMBED_DIM):
          d16 = jnp.full((16,), jnp.int32(d), jnp.int32)
          val = plsc.load_gather(tail, [sti, d16])
          plsc.store_scatter(ev, [e, d16], val, mask=smask)

    return carry

  lax.fori_loop(0, _BPW // 16, patch, 0)

  onehots = [
      (lanes == i).astype(jnp.float32) for i in range(16)
  ]

  def quad(g, carry):
    ovec = jnp.zeros((16,), jnp.float32)
    for q in range(4):
      e = g * 4 + q
      clo = c_v[e, pl.ds(0, 16)]
      chi = c_v[e, pl.ds(16, 16)]
      for t, ev in enumerate((e0_v, e1_v, e2_v, e3_v)):
        prod = clo * ev[e, pl.ds(0, 16)] + chi * ev[e, pl.ds(16, 16)]
        ovec = ovec + jnp.sum(prod) * onehots[q * 4 + t]
    out_flat[pl.ds(g * 16, 16)] = ovec
    return carry

  lax.fori_loop(0, _BPW // 4, quad, 0)
  pltpu.sync_copy(out_flat, out_hbm.at[pl.ds(base * 4, _BPW * 4)])


@jax.jit
def kernel(bgc_idx, domain_idx, context_table, domain_table_0,
           domain_table_85, domain_table_170, domain_table_255):
  bgc = bgc_idx.astype(jnp.int32)
  dom = domain_idx.astype(jnp.int32)
  tables = (domain_table_0, domain_table_85, domain_table_170,
            domain_table_255)
  ctx_t = context_table.T.reshape(4, 8, NUM_BGCS)
  tbl_t = tuple(t.T.reshape(4, 8, NUM_DOMAINS) for t in tables)
  ctx_tail = context_table[_CTX_ROWS:]
  dom_tails = tuple(t[_DOM_ROWS:] for t in tables)

  k1 = pl.kernel(
      _k1_body,
      mesh=plsc.VectorSubcoreMesh(core_axis_name="c", subcore_axis_name="s"),
      out_type=(
          tuple(jax.ShapeDtypeStruct((_NW * _CAP // 4, 128), jnp.float32)
                for _ in range(5))
          + tuple(jax.ShapeDtypeStruct((_NW * _NSC, 128), jnp.int32)
                  for _ in range(5))
          + (jax.ShapeDtypeStruct((_NW, 16), jnp.int32),)),
      compiler_params=pltpu.CompilerParams(
          needs_layout_passes=False, use_tc_tiling_on_sc=True),
      scratch_types=[
          pltpu.VMEM((BATCH,), jnp.int32),
          pltpu.VMEM((_LCAP,), jnp.int32),
          pltpu.VMEM((_LCAP,), jnp.int32),
          pltpu.VMEM((2, 4, 8, 1024), jnp.float32),
          pltpu.VMEM((_CAP // 4, 128), jnp.float32),
          pltpu.VMEM((_NSC, 128), jnp.int32),
          pltpu.VMEM((_WCAP,), jnp.int32),
          pltpu.VMEM((_WCAP,), jnp.int32),
          pltpu.VMEM((16,), jnp.int32),
          pltpu.SemaphoreType.DMA,
          pltpu.SemaphoreType.DMA,
      ],
  )
  rows_maps = k1(bgc, dom, ctx_t, *tbl_t)
  rows_maps = tuple(
      r.reshape(_NW * _CAP, EMBED_DIM) if i < 5 else r
      for i, r in enumerate(rows_maps))

  k1b = pl.kernel(
      _k1b_body,
      mesh=plsc.VectorSubcoreMesh(core_axis_name="c", subcore_axis_name="s"),
      out_type=tuple(jax.ShapeDtypeStruct((_G_ROWS, EMBED_DIM), jnp.float32)
                     for _ in range(5)),
      compiler_params=pltpu.CompilerParams(
          needs_layout_passes=False, use_tc_tiling_on_sc=False),
      scratch_types=[
          pltpu.VMEM((_CAP, EMBED_DIM), jnp.float32),
          pltpu.VMEM((_NSC, 128), jnp.int32),
          pltpu.VMEM((16,), jnp.int32),
          pltpu.SemaphoreType.DMA,
      ],
  )
  cg, eg0, eg1, eg2, eg3 = k1b(*rows_maps)

  k2 = pl.kernel(
      _k2_body,
      mesh=plsc.VectorSubcoreMesh(core_axis_name="c", subcore_axis_name="s"),
      out_type=jax.ShapeDtypeStruct((BATCH * 4,), jnp.float32),
      compiler_params=pltpu.CompilerParams(
          needs_layout_passes=False, use_tc_tiling_on_sc=False),
      scratch_types=[
          pltpu.VMEM((_BPW,), jnp.int32),
          pltpu.VMEM((_BPW,), jnp.int32),
          pltpu.VMEM((_BPW, EMBED_DIM), jnp.float32),
          pltpu.VMEM((_BPW, EMBED_DIM), jnp.float32),
          pltpu.VMEM((_BPW, EMBED_DIM), jnp.float32),
          pltpu.VMEM((_BPW, EMBED_DIM), jnp.float32),
          pltpu.VMEM((_BPW, EMBED_DIM), jnp.float32),
          pltpu.VMEM((64, EMBED_DIM), jnp.float32),
          pltpu.VMEM((32, EMBED_DIM), jnp.float32),
          pltpu.VMEM((32, EMBED_DIM), jnp.float32),
          pltpu.VMEM((32, EMBED_DIM), jnp.float32),
          pltpu.VMEM((32, EMBED_DIM), jnp.float32),
          pltpu.VMEM((_BPW * 4,), jnp.float32),
          pltpu.SemaphoreType.DMA,
      ],
  )
  out = k2(bgc, dom, cg, eg0, eg1, eg2, eg3, ctx_tail, *dom_tails)
  return out.reshape(BATCH, 4)


# speedup vs baseline: 1.2872x; 1.1201x over previous
import functools

import jax
import jax.numpy as jnp
from jax import lax
from jax.experimental import pallas as pl
from jax.experimental.pallas import tpu as pltpu
from jax.experimental.pallas import tpu_sc as plsc

EMBED_DIM = 32
BATCH = 16384
NUM_BGCS = 1000000
NUM_DOMAINS = 100000

_INFO = plsc.get_sparse_core_info()
_NC = _INFO.num_cores
_NS = _INFO.num_subcores
_NW = _NC * _NS
_BPW = BATCH // _NW

_CTX_FULL = NUM_BGCS // 128
_DOM_FULL = NUM_DOMAINS // 128
_CTX_ROWS = _CTX_FULL * 128
_DOM_ROWS = _DOM_FULL * 128
_CTX_CPT = 256
_DOM_CPT = 25
_CAP = 1024
_NSC = _CAP // 128
_LCAP = 1344
_WCAP = 192
_PAD_ROWS = 128
_G_ROWS = BATCH + _PAD_ROWS

_LANES = None


def _iota16():
  return jax.lax.iota(jnp.int32, 16)


def _filter_range(idx_v, lo, hi, lst_rb, lst_b):
  lanes = _iota16()

  def body(g, cur):
    for h in range(2):
      rbv = idx_v[pl.ds(g * 32 + h * 16, 16)]
      bv = g * 32 + h * 16 + lanes
      m = (rbv >= lo) & (rbv < hi)
      plsc.store_compressed(lst_rb.at[pl.ds(cur, 16)], rbv, mask=m)
      plsc.store_compressed(lst_b.at[pl.ds(cur, 16)], bv, mask=m)
      cur = cur + jnp.sum(m.astype(jnp.int32))
    return cur

  n = lax.fori_loop(0, BATCH // 32, body, jnp.int32(0))
  n = jnp.minimum(n, _LCAP - 16)
  lst_rb[pl.ds(n, 16)] = jnp.full((16,), jnp.int32(1 << 29), jnp.int32)
  lst_b[pl.ds(n, 16)] = jnp.full((16,), jnp.int32(BATCH), jnp.int32)
  return n


def _phase(tbl_hbm, rows_hbm, maps_hbm, wid, lo_col, hi_col, nwin, wcols,
           last_col, n_lst, lst_rb, lst_b, buf, rowbuf, rb_b, wrb, wb,
           semA, semB, phase_id, cnt_v):
  lanes = _iota16()
  padv = jnp.int32(BATCH) + ((wid * 37 + phase_id * 53 + lanes * 11) & 127)

  def prefill(i, c):
    for h in range(8):
      rb_b[i, pl.ds(h * 16, 16)] = padv
    return c
  lax.fori_loop(0, _NSC, prefill, 0)

  w128 = wcols * 128

  def fire(win, p):
    c0 = jnp.minimum(lo_col + win * wcols, last_col - wcols + 1)
    pltpu.async_copy(tbl_hbm.at[:, :, pl.ds(c0 * 128, w128)],
                     buf.at[p, :, :, pl.ds(0, w128)],
                     semA if p == 0 else semB)

  def drain(p):
    pltpu.make_async_copy(tbl_hbm.at[:, :, pl.ds(0, w128)],
                          buf.at[p, :, :, pl.ds(0, w128)],
                          semA if p == 0 else semB).wait()

  lgroups = (n_lst + 15) // 16

  def wfilter(win):
    c0 = jnp.minimum(lo_col + win * wcols, last_col - wcols + 1)

    def fbody(g, wcur):
      rv = lst_rb[pl.ds(g * 16, 16)]
      bv = lst_b[pl.ds(g * 16, 16)]
      u = (rv >> 7) - c0
      m = (u >= 0) & (u < wcols)
      plsc.store_compressed(wrb.at[pl.ds(wcur, 16)], rv, mask=m)
      plsc.store_compressed(wb.at[pl.ds(wcur, 16)], bv, mask=m)
      return wcur + jnp.sum(m.astype(jnp.int32))

    nw = lax.fori_loop(0, lgroups, fbody, jnp.int32(0))
    nw = jnp.minimum(nw, _WCAP - 16)
    wrb[pl.ds(nw, 16)] = jnp.full((16,), jnp.int32(c0 * 128), jnp.int32)
    wb[pl.ds(nw, 16)] = padv
    return nw

  def extract(win, p, nw, cursor):
    c0 = jnp.minimum(lo_col + win * wcols, last_col - wcols + 1)

    def ebody(j, c):
      rv = wrb[pl.ds(j * 16, 16)]
      bv = wb[pl.ds(j * 16, 16)]
      colv = rv - c0 * 128
      jv = jnp.minimum(c + j * 16 + lanes, _CAP - 1)
      jhi = jv >> 2
      jlo = (jv & 3) * EMBED_DIM
      p16 = jnp.full((16,), jnp.int32(p), jnp.int32)
      for d in range(EMBED_DIM):
        dl = (jnp.int32(d) + lanes) & 31
        val = plsc.load_gather(buf, [p16, dl >> 3, dl & 7, colv])
        plsc.store_scatter(rowbuf, [jhi, jlo + dl], val)
      plsc.store_scatter(rb_b, [jv >> 7, jv & 127], bv)
      return c

    lax.fori_loop(0, (nw + 15) // 16, ebody, cursor)
    return jnp.minimum(cursor + nw, _CAP - 16)

  fire(jnp.int32(0), 0)

  def pair(i, cursor):
    fire(2 * i + 1, 1)
    nw = wfilter(2 * i)
    drain(0)
    cursor = extract(2 * i, 0, nw, cursor)
    fire(2 * i + 2, 0)
    nw = wfilter(2 * i + 1)
    drain(1)
    cursor = extract(2 * i + 1, 1, nw, cursor)
    return cursor

  total = lax.fori_loop(0, nwin // 2, pair, jnp.int32(0))
  drain(0)

  plsc.store_scatter(cnt_v, [jnp.full((16,), jnp.int32(phase_id), jnp.int32)],
                     jnp.full((16,), 0, jnp.int32) + total)
  pltpu.sync_copy(rowbuf, rows_hbm.at[pl.ds(wid * (_CAP // 4), _CAP // 4)])
  pltpu.sync_copy(rb_b, maps_hbm.at[pl.ds(wid * _NSC, _NSC)])


def _k1_body(bgc_hbm, dom_hbm, ctx_hbm, t0_hbm, t1_hbm, t2_hbm, t3_hbm,
             cr_hbm, e0r_hbm, e1r_hbm, e2r_hbm, e3r_hbm,
             cm_hbm, e0m_hbm, e1m_hbm, e2m_hbm, e3m_hbm, cnt_hbm,
             idx_v, lst_rb, lst_b, buf, rowbuf, rb_b, wrb, wb, cnt_v,
             semA, semB):
  wid = lax.axis_index("s") * _NC + lax.axis_index("c")

  pltpu.sync_copy(bgc_hbm, idx_v)
  lo_col = wid * _CTX_CPT
  hi_col = jnp.minimum(lo_col + _CTX_CPT, _CTX_FULL)
  n = _filter_range(idx_v, lo_col * 128, hi_col * 128, lst_rb, lst_b)
  _phase(ctx_hbm, cr_hbm, cm_hbm, wid, lo_col, hi_col, _CTX_CPT // 8, 8,
         jnp.int32(_CTX_FULL - 1), n, lst_rb, lst_b, buf, rowbuf, rb_b,
         wrb, wb, semA, semB, 0, cnt_v)

  pltpu.sync_copy(dom_hbm, idx_v)
  lo_col = wid * _DOM_CPT
  hi_col = jnp.minimum(lo_col + _DOM_CPT, _DOM_FULL)
  n = _filter_range(idx_v, lo_col * 128, hi_col * 128, lst_rb, lst_b)
  for t, (tbl, rows, maps) in enumerate((
      (t0_hbm, e0r_hbm, e0m_hbm), (t1_hbm, e1r_hbm, e1m_hbm),
      (t2_hbm, e2r_hbm, e2m_hbm), (t3_hbm, e3r_hbm, e3m_hbm))):
    _phase(tbl, rows, maps, wid, lo_col, hi_col, 8, 4,
           jnp.int32(_DOM_FULL - 1), n, lst_rb, lst_b, buf, rowbuf, rb_b,
           wrb, wb, semA, semB, 1 + t, cnt_v)
  pltpu.sync_copy(cnt_v, cnt_hbm.at[wid])


def _k1b_body(cr_hbm, e0r_hbm, e1r_hbm, e2r_hbm, e3r_hbm,
              cm_hbm, e0m_hbm, e1m_hbm, e2m_hbm, e3m_hbm, cnt_hbm,
              cg_hbm, eg0_hbm, eg1_hbm, eg2_hbm, eg3_hbm,
              rows_v, map_v, cnt_v, sem):
  wid = lax.axis_index("s") * _NC + lax.axis_index("c")
  lanes = _iota16()
  pltpu.sync_copy(cnt_hbm.at[wid], cnt_v)
  cv = cnt_v[pl.ds(0, 16)]
  for t, (rows, maps, out) in enumerate((
      (cr_hbm, cm_hbm, cg_hbm),
      (e0r_hbm, e0m_hbm, eg0_hbm),
      (e1r_hbm, e1m_hbm, eg1_hbm),
      (e2r_hbm, e2m_hbm, eg2_hbm),
      (e3r_hbm, e3m_hbm, eg3_hbm))):
    cnt = jnp.sum(jnp.where(lanes == t, cv, 0))
    pltpu.sync_copy(rows.at[pl.ds(wid * _CAP, _CAP)], rows_v)
    pltpu.sync_copy(maps.at[pl.ds(wid * _NSC, _NSC)], map_v)
    for i in range(_NSC):
      @pl.when(cnt > i * 128)
      def _():
        pltpu.async_copy(
            rows_v.at[pl.ds(i * 128, 128)], out.at[map_v.at[i]], sem)
    for i in range(_NSC):
      @pl.when(cnt > i * 128)
      def _():
        pltpu.make_async_copy(
            rows_v.at[pl.ds(i * 128, 128)], out.at[map_v.at[i]], sem).wait()


def _k2_body(bgc_hbm, dom_hbm, cg_hbm, eg0_hbm, eg1_hbm, eg2_hbm, eg3_hbm,
             ctail_hbm, dt0_hbm, dt1_hbm, dt2_hbm, dt3_hbm, out_hbm,
             bgc_v, dom_v, c_v, e0_v, e1_v, e2_v, e3_v,
             ctail_v, dt0_v, dt1_v, dt2_v, dt3_v, out_flat, sem):
  wid = lax.axis_index("s") * _NC + lax.axis_index("c")
  base = wid * _BPW
  lanes = _iota16()

  pltpu.sync_copy(bgc_hbm.at[pl.ds(base, _BPW)], bgc_v)
  pltpu.sync_copy(dom_hbm.at[pl.ds(base, _BPW)], dom_v)
  pltpu.sync_copy(ctail_hbm, ctail_v)
  for src, dst in ((dt0_hbm, dt0_v), (dt1_hbm, dt1_v),
                   (dt2_hbm, dt2_v), (dt3_hbm, dt3_v)):
    pltpu.sync_copy(src, dst)
  copies = [
      pltpu.async_copy(cg_hbm.at[pl.ds(base, _BPW)], c_v, sem),
      pltpu.async_copy(eg0_hbm.at[pl.ds(base, _BPW)], e0_v, sem),
      pltpu.async_copy(eg1_hbm.at[pl.ds(base, _BPW)], e1_v, sem),
      pltpu.async_copy(eg2_hbm.at[pl.ds(base, _BPW)], e2_v, sem),
      pltpu.async_copy(eg3_hbm.at[pl.ds(base, _BPW)], e3_v, sem),
  ]
  for cp in copies:
    cp.wait()

  def patch(g, carry):
    e = g * 16 + lanes
    rbv = bgc_v[pl.ds(g * 16, 16)]
    sbv = dom_v[pl.ds(g * 16, 16)]
    cmask = rbv >= _CTX_ROWS
    smask = sbv >= _DOM_ROWS
    cti = jnp.maximum(rbv - _CTX_ROWS, 0)
    sti = jnp.maximum(sbv - _DOM_ROWS, 0)

    @pl.when(jnp.sum(cmask.astype(jnp.int32)) > 0)
    def _():
      for d in range(EMBED_DIM):
        d16 = jnp.full((16,), jnp.int32(d), jnp.int32)
        val = plsc.load_gather(ctail_v, [cti, d16])
        plsc.store_scatter(c_v, [e, d16], val, mask=cmask)

    @pl.when(jnp.sum(smask.astype(jnp.int32)) > 0)
    def _():
      for tail, ev in ((dt0_v, e0_v), (dt1_v, e1_v),
                       (dt2_v, e2_v), (dt3_v, e3_v)):
        for d in range(EMBED_DIM):
          d16 = jnp.full((16,), jnp.int32(d), jnp.int32)
          val = plsc.load_gather(tail, [sti, d16])
          plsc.store_scatter(ev, [e, d16], val, mask=smask)

    return carry

  lax.fori_loop(0, _BPW // 16, patch, 0)

  onehots = [
      (lanes == i).astype(jnp.float32) for i in range(16)
  ]

  def quad(g, carry):
    ovec = jnp.zeros((16,), jnp.float32)
    for q in range(4):
      e = g * 4 + q
      clo = c_v[e, pl.ds(0, 16)]
      chi = c_v[e, pl.ds(16, 16)]
      for t, ev in enumerate((e0_v, e1_v, e2_v, e3_v)):
        prod = clo * ev[e, pl.ds(0, 16)] + chi * ev[e, pl.ds(16, 16)]
        ovec = ovec + jnp.sum(prod) * onehots[q * 4 + t]
    out_flat[pl.ds(g * 16, 16)] = ovec
    return carry

  lax.fori_loop(0, _BPW // 4, quad, 0)
  pltpu.sync_copy(out_flat, out_hbm.at[pl.ds(base * 4, _BPW * 4)])


@jax.jit
def kernel(bgc_idx, domain_idx, context_table, domain_table_0,
           domain_table_85, domain_table_170, domain_table_255):
  bgc = bgc_idx.astype(jnp.int32)
  dom = domain_idx.astype(jnp.int32)
  tables = (domain_table_0, domain_table_85, domain_table_170,
            domain_table_255)
  ctx_t = context_table.T.reshape(4, 8, NUM_BGCS)
  tbl_t = tuple(t.T.reshape(4, 8, NUM_DOMAINS) for t in tables)
  ctx_tail = context_table[_CTX_ROWS:]
  dom_tails = tuple(t[_DOM_ROWS:] for t in tables)

  k1 = pl.kernel(
      _k1_body,
      mesh=plsc.VectorSubcoreMesh(core_axis_name="c", subcore_axis_name="s"),
      out_type=(
          tuple(jax.ShapeDtypeStruct((_NW * _CAP // 4, 128), jnp.float32)
                for _ in range(5))
          + tuple(jax.ShapeDtypeStruct((_NW * _NSC, 128), jnp.int32)
                  for _ in range(5))
          + (jax.ShapeDtypeStruct((_NW, 16), jnp.int32),)),
      compiler_params=pltpu.CompilerParams(
          needs_layout_passes=False, use_tc_tiling_on_sc=True),
      scratch_types=[
          pltpu.VMEM((BATCH,), jnp.int32),
          pltpu.VMEM((_LCAP,), jnp.int32),
          pltpu.VMEM((_LCAP,), jnp.int32),
          pltpu.VMEM((2, 4, 8, 1024), jnp.float32),
          pltpu.VMEM((_CAP // 4, 128), jnp.float32),
          pltpu.VMEM((_NSC, 128), jnp.int32),
          pltpu.VMEM((_WCAP,), jnp.int32),
          pltpu.VMEM((_WCAP,), jnp.int32),
          pltpu.VMEM((16,), jnp.int32),
          pltpu.SemaphoreType.DMA,
          pltpu.SemaphoreType.DMA,
      ],
  )
  rows_maps = k1(bgc, dom, ctx_t, *tbl_t)
  rows_maps = tuple(
      r.reshape(_NW * _CAP, EMBED_DIM) if i < 5 else r
      for i, r in enumerate(rows_maps))

  k1b = pl.kernel(
      _k1b_body,
      mesh=plsc.VectorSubcoreMesh(core_axis_name="c", subcore_axis_name="s"),
      out_type=tuple(jax.ShapeDtypeStruct((_G_ROWS, EMBED_DIM), jnp.float32)
                     for _ in range(5)),
      compiler_params=pltpu.CompilerParams(
          needs_layout_passes=False, use_tc_tiling_on_sc=False),
      scratch_types=[
          pltpu.VMEM((_CAP, EMBED_DIM), jnp.float32),
          pltpu.VMEM((_NSC, 128), jnp.int32),
          pltpu.VMEM((16,), jnp.int32),
          pltpu.SemaphoreType.DMA,
      ],
  )
  cg, eg0, eg1, eg2, eg3 = k1b(*rows_maps)

  k2 = pl.kernel(
      _k2_body,
      mesh=plsc.VectorSubcoreMesh(core_axis_name="c", subcore_axis_name="s"),
      out_type=jax.ShapeDtypeStruct((BATCH * 4,), jnp.float32),
      compiler_params=pltpu.CompilerParams(
          needs_layout_passes=False, use_tc_tiling_on_sc=False),
      scratch_types=[
          pltpu.VMEM((_BPW,), jnp.int32),
          pltpu.VMEM((_BPW,), jnp.int32),
          pltpu.VMEM((_BPW, EMBED_DIM), jnp.float32),
          pltpu.VMEM((_BPW, EMBED_DIM), jnp.float32),
          pltpu.VMEM((_BPW, EMBED_DIM), jnp.float32),
          pltpu.VMEM((_BPW, EMBED_DIM), jnp.float32),
          pltpu.VMEM((_BPW, EMBED_DIM), jnp.float32),
          pltpu.VMEM((64, EMBED_DIM), jnp.float32),
          pltpu.VMEM((32, EMBED_DIM), jnp.float32),
          pltpu.VMEM((32, EMBED_DIM), jnp.float32),
          pltpu.VMEM((32, EMBED_DIM), jnp.float32),
          pltpu.VMEM((32, EMBED_DIM), jnp.float32),
          pltpu.VMEM((_BPW * 4,), jnp.float32),
          pltpu.SemaphoreType.DMA,
      ],
  )
  out = k2(bgc, dom, cg, eg0, eg1, eg2, eg3, ctx_tail, *dom_tails)
  return out.reshape(BATCH, 4)
